# baseline (speedup 1.0000x reference)
import jax
import jax.numpy as jnp
import numpy as np
from jax.experimental import pallas as pl


def _build_d() -> np.ndarray:
    d = np.zeros((32, 63), dtype=np.float32)
    for t in range(3):
        for j in range(10):
            col = 21 * t + 10 + j
            d[10 * t + j, col] = -1.0
            if j == 0:
                d[30, col] = 1.0
            else:
                d[10 * t + (j - 1), col] = 1.0
    return d


_D = _build_d()


def _angles_kernel(x_ref, idx_ref, thr_ref, d_ref, out_ref):
    x = x_ref[0]
    idx = idx_ref[0]
    n = x.shape[0]

    c9 = x[:, 0:9]
    zero_row9 = jnp.zeros((1, 9), dtype=jnp.float32)
    c9s = jnp.concatenate([c9[1:, :], zero_row9], axis=0)

    d_idx = idx[1:, :] - idx[:-1, :]
    good_col = jnp.concatenate(
        [jnp.where(d_idx == 1, 1.0, 0.0).astype(jnp.float32),
         jnp.zeros((1, 1), dtype=jnp.float32)], axis=0)

    m = jnp.concatenate([c9, good_col], axis=1)
    mt = jnp.transpose(m)
    xst = jnp.transpose(c9s[:, 0:6])

    N, CA, C = mt[0:3], mt[3:6], mt[6:9]
    good = mt[9:10] > 0.5
    Nn, CAn = xst[0:3], xst[3:6]

    def cosine(v1, v2):
        dot = jnp.sum(v1 * v2, axis=0, keepdims=True)
        n1 = jnp.sqrt(jnp.sum(v1 * v1, axis=0, keepdims=True))
        n2 = jnp.sqrt(jnp.sum(v2 * v2, axis=0, keepdims=True))
        c = dot / (n1 * n2 + 1e-10)
        return jnp.clip(c, -1.0 + 1e-7, 1.0 - 1e-7)

    c1 = cosine(N - CA, C - CA)
    c2 = jnp.where(good, cosine(CA - C, Nn - C), 2.0)
    c3 = jnp.where(good, cosine(C - Nn, CAn - Nn), 2.0)

    row = jax.lax.broadcasted_iota(jnp.int32, (32, n), 0)
    cb = jnp.where(row < 10, jnp.broadcast_to(c1, (32, n)),
                   jnp.where(row < 20, jnp.broadcast_to(c2, (32, n)),
                             jnp.broadcast_to(c3, (32, n))))
    thr = thr_ref[:, 0:1]
    u_t = jnp.where(cb < thr, 1.0, 0.0).astype(jnp.float32)

    feats = jax.lax.dot_general(
        u_t, d_ref[...],
        dimension_numbers=(((0,), (0,)), ((), ())),
        preferred_element_type=jnp.float32)
    out_ref[0] = feats


@jax.jit
def kernel(coords, mask, residue_pdb_idx):
    del mask
    b, n = coords.shape[0], coords.shape[1]
    x = coords.reshape(b, n, 111)
    idx3 = residue_pdb_idx.astype(jnp.int32).reshape(b, n, 1)

    limits = jnp.linspace(-jnp.pi, jnp.pi, 20)
    thr10 = jnp.cos(limits[10:])
    thr32 = jnp.concatenate(
        [jnp.tile(thr10, 3), jnp.array([4.0, -4.0], dtype=jnp.float32)])
    thr = jnp.broadcast_to(thr32[:, None], (32, 128))
    d = jnp.asarray(_D)

    out = pl.pallas_call(
        _angles_kernel,
        grid=(b,),
        in_specs=[
            pl.BlockSpec((1, n, 111), lambda i: (i, 0, 0)),
            pl.BlockSpec((1, n, 1), lambda i: (i, 0, 0)),
            pl.BlockSpec((32, 128), lambda i: (0, 0)),
            pl.BlockSpec((32, 63), lambda i: (0, 0)),
        ],
        out_specs=pl.BlockSpec((1, n, 63), lambda i: (i, 0, 0)),
        out_shape=jax.ShapeDtypeStruct((b, n, 63), jnp.float32),
    )(x, idx3, thr, d)
    return out
